# baseline (speedup 1.0000x reference)
import functools

import jax
import jax.numpy as jnp
from jax import lax
from jax.experimental import pallas as pl
from jax.experimental.pallas import tpu as pltpu
from jax.experimental.pallas import tpu_sc as plsc

_INFO = plsc.get_sparse_core_info()
_NC, _NS = _INFO.num_cores, _INFO.num_subcores
_NW = _NC * _NS


def _make_gather(V, D, BATCH, SEQ, chunk):
    B = BATCH * SEQ
    assert B % _NW == 0
    b_per_w = B // _NW
    assert SEQ % b_per_w == 0
    w_per_row = SEQ // b_per_w
    assert b_per_w % chunk == 0
    n_chunks = b_per_w // chunk
    nbuf = 4
    assert n_chunks % nbuf == 0
    mesh = plsc.VectorSubcoreMesh(core_axis_name="c", subcore_axis_name="s")

    @functools.partial(
        pl.kernel,
        mesh=mesh,
        out_type=jax.ShapeDtypeStruct((B, D), jnp.float32),
        scratch_types=[
            pltpu.VMEM((b_per_w,), jnp.int32),
        ] + [pltpu.VMEM((chunk, D), jnp.float32)] * nbuf + [
            pltpu.SemaphoreType.DMA,
            pltpu.SemaphoreType.DMA,
        ] + [pltpu.SemaphoreType.DMA] * (2 * nbuf),
    )
    def gather_kernel(table_hbm, ids_hbm, out_hbm, idx_v, *rest):
        bufs = rest[:nbuf]
        isem = rest[nbuf]
        isem2 = rest[nbuf + 1]
        gsems = rest[nbuf + 2:2 * nbuf + 2]
        wsems = rest[2 * nbuf + 2:]
        wid = lax.axis_index("s") * _NC + lax.axis_index("c")
        base = wid * b_per_w
        row = wid // w_per_row
        col = (wid % w_per_row) * b_per_w
        lead = 128
        head = pltpu.make_async_copy(
            ids_hbm.at[row, pl.ds(col, lead)], idx_v.at[pl.ds(0, lead)],
            isem)
        tail = pltpu.make_async_copy(
            ids_hbm.at[row, pl.ds(col + lead, b_per_w - lead)],
            idx_v.at[pl.ds(lead, b_per_w - lead)], isem2)
        head.start()
        tail.start()

        def gather(g, b):
            pltpu.async_copy(
                table_hbm.at[idx_v.at[pl.ds(g * chunk, chunk)]], bufs[b],
                gsems[b])

        def wait_gather(g, b):
            pltpu.make_async_copy(
                table_hbm.at[idx_v.at[pl.ds(g * chunk, chunk)]], bufs[b],
                gsems[b]).wait()

        def write(g, b):
            pltpu.async_copy(
                bufs[b], out_hbm.at[pl.ds(base + g * chunk, chunk)], wsems[b])

        def wait_write(g, b):
            pltpu.make_async_copy(
                bufs[b], out_hbm.at[pl.ds(base + g * chunk, chunk)],
                wsems[b]).wait()

        head.wait()
        for b in range(nbuf):
            gather(b, b)
        tail.wait()

        def body(h, carry):
            g = h * nbuf
            for b in range(nbuf):
                wait_gather(g + b, b)
                write(g + b, b)
            for b in range(nbuf):
                wait_write(g + b, b)
                gather(g + nbuf + b, b)
            return carry

        lax.fori_loop(0, n_chunks // nbuf - 1, body, 0)
        g_last = n_chunks - nbuf
        for b in range(nbuf):
            wait_gather(g_last + b, b)
            write(g_last + b, b)
        for b in range(nbuf):
            wait_write(g_last + b, b)

    return gather_kernel


def kernel(input_ids, attention_mask, embed_table):
    del attention_mask
    V, D = embed_table.shape
    BATCH, SEQ = input_ids.shape
    out = _make_gather(V, D, BATCH, SEQ, 16)(embed_table, input_ids)
    return out.reshape(BATCH, SEQ, D)


# speedup vs baseline: 1.0147x; 1.0147x over previous
import functools

import jax
import jax.numpy as jnp
from jax import lax
from jax.experimental import pallas as pl
from jax.experimental.pallas import tpu as pltpu
from jax.experimental.pallas import tpu_sc as plsc

_INFO = plsc.get_sparse_core_info()
_NC, _NS = _INFO.num_cores, _INFO.num_subcores
_NW = _NC * _NS


def _make_gather(V, D, BATCH, SEQ, chunk):
    B = BATCH * SEQ
    assert B % _NW == 0
    b_per_w = B // _NW
    assert SEQ % b_per_w == 0
    w_per_row = SEQ // b_per_w
    assert b_per_w % chunk == 0
    n_chunks = b_per_w // chunk
    nbuf = 8
    assert n_chunks % nbuf == 0
    mesh = plsc.VectorSubcoreMesh(core_axis_name="c", subcore_axis_name="s")

    @functools.partial(
        pl.kernel,
        mesh=mesh,
        out_type=jax.ShapeDtypeStruct((B, D), jnp.float32),
        scratch_types=[
            pltpu.VMEM((b_per_w,), jnp.int32),
        ] + [pltpu.VMEM((chunk, D), jnp.float32)] * nbuf + [
            pltpu.SemaphoreType.DMA,
            pltpu.SemaphoreType.DMA,
        ] + [pltpu.SemaphoreType.DMA] * (2 * nbuf),
    )
    def gather_kernel(table_hbm, ids_hbm, out_hbm, idx_v, *rest):
        bufs = rest[:nbuf]
        isem = rest[nbuf]
        isem2 = rest[nbuf + 1]
        gsems = rest[nbuf + 2:2 * nbuf + 2]
        wsems = rest[2 * nbuf + 2:]
        wid = lax.axis_index("s") * _NC + lax.axis_index("c")
        base = wid * b_per_w
        row = wid // w_per_row
        col = (wid % w_per_row) * b_per_w
        lead = 128
        head = pltpu.make_async_copy(
            ids_hbm.at[row, pl.ds(col, lead)], idx_v.at[pl.ds(0, lead)],
            isem)
        tail = pltpu.make_async_copy(
            ids_hbm.at[row, pl.ds(col + lead, b_per_w - lead)],
            idx_v.at[pl.ds(lead, b_per_w - lead)], isem2)
        head.start()
        tail.start()

        def gather(g, b):
            pltpu.async_copy(
                table_hbm.at[idx_v.at[pl.ds(g * chunk, chunk)]], bufs[b],
                gsems[b])

        def wait_gather(g, b):
            pltpu.make_async_copy(
                table_hbm.at[idx_v.at[pl.ds(g * chunk, chunk)]], bufs[b],
                gsems[b]).wait()

        def write(g, b):
            pltpu.async_copy(
                bufs[b], out_hbm.at[pl.ds(base + g * chunk, chunk)], wsems[b])

        def wait_write(g, b):
            pltpu.make_async_copy(
                bufs[b], out_hbm.at[pl.ds(base + g * chunk, chunk)],
                wsems[b]).wait()

        head.wait()
        for b in range(nbuf):
            gather(b, b)
        tail.wait()

        def body(h, carry):
            g = h * nbuf
            for b in range(nbuf):
                wait_gather(g + b, b)
                write(g + b, b)
            for b in range(nbuf):
                wait_write(g + b, b)
                gather(g + nbuf + b, b)
            return carry

        lax.fori_loop(0, n_chunks // nbuf - 1, body, 0)
        g_last = n_chunks - nbuf
        for b in range(nbuf):
            wait_gather(g_last + b, b)
            write(g_last + b, b)
        for b in range(nbuf):
            wait_write(g_last + b, b)

    return gather_kernel


def kernel(input_ids, attention_mask, embed_table):
    del attention_mask
    V, D = embed_table.shape
    BATCH, SEQ = input_ids.shape
    out = _make_gather(V, D, BATCH, SEQ, 8)(embed_table, input_ids)
    return out.reshape(BATCH, SEQ, D)
